# Initial kernel scaffold; baseline (speedup 1.0000x reference)
import jax
import jax.numpy as jnp
from jax.experimental import pallas as pl


def kernel(feat, edge_index, W, b):
    raise NotImplementedError("write your pallas kernel here")


# speedup vs baseline: 3.9316x; 3.9316x over previous
import functools

import jax
import jax.numpy as jnp
from jax import lax
from jax.experimental import pallas as pl
from jax.experimental.pallas import tpu as pltpu
from jax.experimental.pallas import tpu_sc as plsc

N = 10000
E = 320000
D = 128
NC = 2
NS = 16
NW = NC * NS
EPW = E // NW
CH = 80
NCHUNK = EPW // CH
RPT = 624
TAIL = N - NS * RPT
TBASE = NS * RPT

_mesh = plsc.VectorSubcoreMesh(core_axis_name="c", subcore_axis_name="s")


@functools.partial(
    pl.kernel,
    out_type=jax.ShapeDtypeStruct((NC, N, D), jnp.float32),
    mesh=_mesh,
    scratch_types=[
        pltpu.VMEM((CH,), jnp.int32),
        pltpu.VMEM((CH, D), jnp.float32),
        pltpu.VMEM_SHARED((N, D), jnp.float32),
    ],
)
def _deg_kernel(dst_hbm, zeros_hbm, ones_hbm, out_hbm, idx_v, ones_v, acc_sh):
    c = lax.axis_index("c")
    s = lax.axis_index("s")
    wid = s * NC + c
    pltpu.sync_copy(zeros_hbm.at[pl.ds(s * RPT, RPT)],
                    acc_sh.at[pl.ds(s * RPT, RPT)])

    @pl.when(s == 0)
    def _():
        pltpu.sync_copy(zeros_hbm.at[pl.ds(TBASE, TAIL)],
                        acc_sh.at[pl.ds(TBASE, TAIL)])

    pltpu.sync_copy(ones_hbm, ones_v)
    plsc.subcore_barrier()

    base = wid * EPW

    def body(i, carry):
        pltpu.sync_copy(dst_hbm.at[pl.ds(base + i * CH, CH)], idx_v)
        pltpu.sync_copy(ones_v, acc_sh.at[idx_v], add=True)
        return carry

    lax.fori_loop(0, NCHUNK, body, 0)
    plsc.subcore_barrier()
    pltpu.sync_copy(acc_sh.at[pl.ds(s * RPT, RPT)],
                    out_hbm.at[c, pl.ds(s * RPT, RPT)])

    @pl.when(s == 0)
    def _():
        pltpu.sync_copy(acc_sh.at[pl.ds(TBASE, TAIL)],
                        out_hbm.at[c, pl.ds(TBASE, TAIL)])


@functools.partial(
    pl.kernel,
    out_type=jax.ShapeDtypeStruct((NC, N, D), jnp.float32),
    mesh=_mesh,
    scratch_types=[
        pltpu.VMEM((CH,), jnp.int32),
        pltpu.VMEM((CH,), jnp.int32),
        pltpu.VMEM((CH, D), jnp.float32),
        pltpu.VMEM_SHARED((N, D), jnp.float32),
        pltpu.SemaphoreType.DMA,
    ],
)
def _hop_kernel(g_hbm, src_hbm, dst_hbm, zeros_hbm, out_hbm,
                sidx_v, didx_v, rows_v, acc_sh, sem):
    c = lax.axis_index("c")
    s = lax.axis_index("s")
    wid = s * NC + c
    pltpu.sync_copy(zeros_hbm.at[pl.ds(s * RPT, RPT)],
                    acc_sh.at[pl.ds(s * RPT, RPT)])

    @pl.when(s == 0)
    def _():
        pltpu.sync_copy(zeros_hbm.at[pl.ds(TBASE, TAIL)],
                        acc_sh.at[pl.ds(TBASE, TAIL)])

    plsc.subcore_barrier()

    base = wid * EPW

    def body(i, carry):
        pltpu.sync_copy(src_hbm.at[pl.ds(base + i * CH, CH)], sidx_v)
        pltpu.sync_copy(dst_hbm.at[pl.ds(base + i * CH, CH)], didx_v)
        pltpu.async_copy(g_hbm.at[sidx_v], rows_v, sem).wait()
        pltpu.sync_copy(rows_v, acc_sh.at[didx_v], add=True)
        return carry

    lax.fori_loop(0, NCHUNK, body, 0)
    plsc.subcore_barrier()
    pltpu.sync_copy(acc_sh.at[pl.ds(s * RPT, RPT)],
                    out_hbm.at[c, pl.ds(s * RPT, RPT)])

    @pl.when(s == 0)
    def _():
        pltpu.sync_copy(acc_sh.at[pl.ds(TBASE, TAIL)],
                        out_hbm.at[c, pl.ds(TBASE, TAIL)])


_BR = 2000


def _norm_body(dacc_ref, feat_ref, g_ref, norm_ref):
    deg = dacc_ref[0, :, 0:1] + dacc_ref[1, :, 0:1]
    deg = jnp.maximum(deg, 1.0)
    nrm = lax.rsqrt(deg)
    norm_ref[...] = nrm
    g_ref[...] = feat_ref[...] * nrm


_norm_call = pl.pallas_call(
    _norm_body,
    grid=(N // _BR,),
    in_specs=[
        pl.BlockSpec((NC, _BR, D), lambda i: (0, i, 0)),
        pl.BlockSpec((_BR, D), lambda i: (i, 0)),
    ],
    out_specs=[
        pl.BlockSpec((_BR, D), lambda i: (i, 0)),
        pl.BlockSpec((_BR, 1), lambda i: (i, 0)),
    ],
    out_shape=[
        jax.ShapeDtypeStruct((N, D), jnp.float32),
        jax.ShapeDtypeStruct((N, 1), jnp.float32),
    ],
)


def _mid_body(p_ref, norm_ref, g_ref):
    nrm = norm_ref[...]
    g_ref[...] = (p_ref[0] + p_ref[1]) * (nrm * nrm)


_mid_call = pl.pallas_call(
    _mid_body,
    grid=(N // _BR,),
    in_specs=[
        pl.BlockSpec((NC, _BR, D), lambda i: (0, i, 0)),
        pl.BlockSpec((_BR, 1), lambda i: (i, 0)),
    ],
    out_specs=pl.BlockSpec((_BR, D), lambda i: (i, 0)),
    out_shape=jax.ShapeDtypeStruct((N, D), jnp.float32),
)


def _fin_body(q_ref, norm_ref, wt_ref, b_ref, out_ref):
    h = (q_ref[0] + q_ref[1]) * norm_ref[...]
    out_ref[...] = (
        jnp.dot(h, wt_ref[...], preferred_element_type=jnp.float32)
        + b_ref[...]
    )


_fin_call = pl.pallas_call(
    _fin_body,
    grid=(N // _BR,),
    in_specs=[
        pl.BlockSpec((NC, _BR, D), lambda i: (0, i, 0)),
        pl.BlockSpec((_BR, 1), lambda i: (i, 0)),
        pl.BlockSpec((D, D), lambda i: (0, 0)),
        pl.BlockSpec((1, D), lambda i: (0, 0)),
    ],
    out_specs=pl.BlockSpec((_BR, D), lambda i: (i, 0)),
    out_shape=jax.ShapeDtypeStruct((N, D), jnp.float32),
)


def kernel(feat, edge_index, W, b):
    ei = edge_index.astype(jnp.int32)
    src = ei[0]
    dst = ei[1]
    zeros = jnp.zeros((N, D), jnp.float32)
    ones = jnp.ones((CH, D), jnp.float32)

    dacc = _deg_kernel(dst, zeros, ones)
    g1, norm = _norm_call(dacc, feat)
    p = _hop_kernel(g1, src, dst, zeros)
    g2 = _mid_call(p, norm)
    q = _hop_kernel(g2, src, dst, zeros)
    out = _fin_call(q, norm, W.T.astype(jnp.float32), b.reshape(1, D))
    return out


# speedup vs baseline: 7.5397x; 1.9177x over previous
import functools

import jax
import jax.numpy as jnp
from jax import lax
from jax.experimental import pallas as pl
from jax.experimental.pallas import tpu as pltpu
from jax.experimental.pallas import tpu_sc as plsc

N = 10000
E = 320000
D = 128
NC = 2
NS = 16
NW = NC * NS
EPW = E // NW
CH = 80
NCHUNK = EPW // CH
RPT = 624
TAIL = N - NS * RPT
TBASE = NS * RPT

_mesh = plsc.VectorSubcoreMesh(core_axis_name="c", subcore_axis_name="s")


@functools.partial(
    pl.kernel,
    out_type=jax.ShapeDtypeStruct((NC, N, D), jnp.float32),
    mesh=_mesh,
    scratch_types=[
        pltpu.VMEM((CH,), jnp.int32),
        pltpu.VMEM((CH, D), jnp.float32),
        pltpu.VMEM_SHARED((N, D), jnp.float32),
    ],
)
def _deg_kernel(dst_hbm, zeros_hbm, ones_hbm, out_hbm, idx_v, ones_v, acc_sh):
    c = lax.axis_index("c")
    s = lax.axis_index("s")
    wid = s * NC + c
    pltpu.sync_copy(zeros_hbm.at[pl.ds(s * RPT, RPT)],
                    acc_sh.at[pl.ds(s * RPT, RPT)])

    @pl.when(s == 0)
    def _():
        pltpu.sync_copy(zeros_hbm.at[pl.ds(TBASE, TAIL)],
                        acc_sh.at[pl.ds(TBASE, TAIL)])

    pltpu.sync_copy(ones_hbm, ones_v)
    plsc.subcore_barrier()

    base = wid * EPW

    def body(i, carry):
        pltpu.sync_copy(dst_hbm.at[pl.ds(base + i * CH, CH)], idx_v)
        pltpu.sync_copy(ones_v, acc_sh.at[idx_v], add=True)
        return carry

    lax.fori_loop(0, NCHUNK, body, 0)
    plsc.subcore_barrier()
    pltpu.sync_copy(acc_sh.at[pl.ds(s * RPT, RPT)],
                    out_hbm.at[c, pl.ds(s * RPT, RPT)])

    @pl.when(s == 0)
    def _():
        pltpu.sync_copy(acc_sh.at[pl.ds(TBASE, TAIL)],
                        out_hbm.at[c, pl.ds(TBASE, TAIL)])


assert NCHUNK == 125


@functools.partial(
    pl.kernel,
    out_type=jax.ShapeDtypeStruct((NC, N, D), jnp.float32),
    mesh=_mesh,
    scratch_types=[
        pltpu.VMEM((EPW,), jnp.int32),
        pltpu.VMEM((CH,), jnp.int32),
        pltpu.VMEM((CH,), jnp.int32),
        pltpu.VMEM((CH, D), jnp.float32),
        pltpu.VMEM((CH, D), jnp.float32),
        pltpu.VMEM_SHARED((N, D), jnp.float32),
        pltpu.SemaphoreType.DMA,
        pltpu.SemaphoreType.DMA,
        pltpu.SemaphoreType.DMA,
        pltpu.SemaphoreType.DMA,
    ],
)
def _hop_kernel(g_hbm, src_hbm, dst_hbm, zeros_hbm, out_hbm,
                sidx_v, didx0, didx1, rows0, rows1, acc_sh,
                sg0, sg1, si0, si1):
    c = lax.axis_index("c")
    s = lax.axis_index("s")
    wid = s * NC + c
    base = wid * EPW

    pltpu.sync_copy(src_hbm.at[pl.ds(base, EPW)], sidx_v)
    pltpu.async_copy(dst_hbm.at[pl.ds(base, CH)], didx0, si0)
    pltpu.async_copy(dst_hbm.at[pl.ds(base + CH, CH)], didx1, si1)
    pltpu.async_copy(g_hbm.at[sidx_v.at[pl.ds(0, CH)]], rows0, sg0)
    pltpu.async_copy(g_hbm.at[sidx_v.at[pl.ds(CH, CH)]], rows1, sg1)

    pltpu.sync_copy(zeros_hbm.at[pl.ds(s * RPT, RPT)],
                    acc_sh.at[pl.ds(s * RPT, RPT)])

    @pl.when(s == 0)
    def _():
        pltpu.sync_copy(zeros_hbm.at[pl.ds(TBASE, TAIL)],
                        acc_sh.at[pl.ds(TBASE, TAIL)])

    plsc.subcore_barrier()

    def body(g, carry):
        a = 2 * g
        pltpu.make_async_copy(dst_hbm.at[pl.ds(base + a * CH, CH)],
                              didx0, si0).wait()
        pltpu.make_async_copy(g_hbm.at[sidx_v.at[pl.ds(a * CH, CH)]],
                              rows0, sg0).wait()
        pltpu.sync_copy(rows0, acc_sh.at[didx0], add=True)
        pltpu.async_copy(dst_hbm.at[pl.ds(base + (a + 2) * CH, CH)],
                         didx0, si0)
        pltpu.async_copy(g_hbm.at[sidx_v.at[pl.ds((a + 2) * CH, CH)]],
                         rows0, sg0)
        pltpu.make_async_copy(dst_hbm.at[pl.ds(base + (a + 1) * CH, CH)],
                              didx1, si1).wait()
        pltpu.make_async_copy(g_hbm.at[sidx_v.at[pl.ds((a + 1) * CH, CH)]],
                              rows1, sg1).wait()
        pltpu.sync_copy(rows1, acc_sh.at[didx1], add=True)

        @pl.when(g < (NCHUNK - 3) // 2)
        def _():
            pltpu.async_copy(dst_hbm.at[pl.ds(base + (a + 3) * CH, CH)],
                             didx1, si1)
            pltpu.async_copy(g_hbm.at[sidx_v.at[pl.ds((a + 3) * CH, CH)]],
                             rows1, sg1)

        return carry

    lax.fori_loop(0, (NCHUNK - 1) // 2, body, 0)

    last = NCHUNK - 1
    pltpu.make_async_copy(dst_hbm.at[pl.ds(base + last * CH, CH)],
                          didx0, si0).wait()
    pltpu.make_async_copy(g_hbm.at[sidx_v.at[pl.ds(last * CH, CH)]],
                          rows0, sg0).wait()
    pltpu.sync_copy(rows0, acc_sh.at[didx0], add=True)

    plsc.subcore_barrier()
    pltpu.sync_copy(acc_sh.at[pl.ds(s * RPT, RPT)],
                    out_hbm.at[c, pl.ds(s * RPT, RPT)])

    @pl.when(s == 0)
    def _():
        pltpu.sync_copy(acc_sh.at[pl.ds(TBASE, TAIL)],
                        out_hbm.at[c, pl.ds(TBASE, TAIL)])


_BR = 2000


def _norm_body(dacc_ref, feat_ref, g_ref, norm_ref):
    deg = dacc_ref[0, :, 0:1] + dacc_ref[1, :, 0:1]
    deg = jnp.maximum(deg, 1.0)
    nrm = lax.rsqrt(deg)
    norm_ref[...] = nrm
    g_ref[...] = feat_ref[...] * nrm


_norm_call = pl.pallas_call(
    _norm_body,
    grid=(N // _BR,),
    in_specs=[
        pl.BlockSpec((NC, _BR, D), lambda i: (0, i, 0)),
        pl.BlockSpec((_BR, D), lambda i: (i, 0)),
    ],
    out_specs=[
        pl.BlockSpec((_BR, D), lambda i: (i, 0)),
        pl.BlockSpec((_BR, 1), lambda i: (i, 0)),
    ],
    out_shape=[
        jax.ShapeDtypeStruct((N, D), jnp.float32),
        jax.ShapeDtypeStruct((N, 1), jnp.float32),
    ],
)


def _mid_body(p_ref, norm_ref, g_ref):
    nrm = norm_ref[...]
    g_ref[...] = (p_ref[0] + p_ref[1]) * (nrm * nrm)


_mid_call = pl.pallas_call(
    _mid_body,
    grid=(N // _BR,),
    in_specs=[
        pl.BlockSpec((NC, _BR, D), lambda i: (0, i, 0)),
        pl.BlockSpec((_BR, 1), lambda i: (i, 0)),
    ],
    out_specs=pl.BlockSpec((_BR, D), lambda i: (i, 0)),
    out_shape=jax.ShapeDtypeStruct((N, D), jnp.float32),
)


def _fin_body(q_ref, norm_ref, wt_ref, b_ref, out_ref):
    h = (q_ref[0] + q_ref[1]) * norm_ref[...]
    out_ref[...] = (
        jnp.dot(h, wt_ref[...], preferred_element_type=jnp.float32)
        + b_ref[...]
    )


_fin_call = pl.pallas_call(
    _fin_body,
    grid=(N // _BR,),
    in_specs=[
        pl.BlockSpec((NC, _BR, D), lambda i: (0, i, 0)),
        pl.BlockSpec((_BR, 1), lambda i: (i, 0)),
        pl.BlockSpec((D, D), lambda i: (0, 0)),
        pl.BlockSpec((1, D), lambda i: (0, 0)),
    ],
    out_specs=pl.BlockSpec((_BR, D), lambda i: (i, 0)),
    out_shape=jax.ShapeDtypeStruct((N, D), jnp.float32),
)


def kernel(feat, edge_index, W, b):
    ei = edge_index.astype(jnp.int32)
    src = ei[0]
    dst = ei[1]
    zeros = jnp.zeros((N, D), jnp.float32)
    ones = jnp.ones((CH, D), jnp.float32)

    dacc = _deg_kernel(dst, zeros, ones)
    g1, norm = _norm_call(dacc, feat)
    p = _hop_kernel(g1, src, dst, zeros)
    g2 = _mid_call(p, norm)
    q = _hop_kernel(g2, src, dst, zeros)
    out = _fin_call(q, norm, W.T.astype(jnp.float32), b.reshape(1, D))
    return out


# speedup vs baseline: 9.1551x; 1.2143x over previous
import functools

import jax
import jax.numpy as jnp
from jax import lax
from jax.experimental import pallas as pl
from jax.experimental.pallas import tpu as pltpu
from jax.experimental.pallas import tpu_sc as plsc

N = 10000
E = 320000
D = 128
NC = 2
NS = 16
NW = NC * NS
EPW = E // NW
CH = 128
NF = EPW // CH
TEDGE = EPW - NF * CH
TOFF = NF * CH
RPT = 624
TAIL = N - NS * RPT
TBASE = NS * RPT

_mesh = plsc.VectorSubcoreMesh(core_axis_name="c", subcore_axis_name="s")


def _zero_acc(zeros_hbm, acc_sh, s):
    pltpu.sync_copy(zeros_hbm.at[pl.ds(s * RPT, RPT)],
                    acc_sh.at[pl.ds(s * RPT, RPT)])

    @pl.when(s == 0)
    def _():
        pltpu.sync_copy(zeros_hbm.at[pl.ds(TBASE, TAIL)],
                        acc_sh.at[pl.ds(TBASE, TAIL)])


def _write_back(acc_sh, out_hbm, c, s):
    pltpu.sync_copy(acc_sh.at[pl.ds(s * RPT, RPT)],
                    out_hbm.at[c, pl.ds(s * RPT, RPT)])

    @pl.when(s == 0)
    def _():
        pltpu.sync_copy(acc_sh.at[pl.ds(TBASE, TAIL)],
                        out_hbm.at[c, pl.ds(TBASE, TAIL)])


@functools.partial(
    pl.kernel,
    out_type=jax.ShapeDtypeStruct((NC, N, D), jnp.float32),
    mesh=_mesh,
    scratch_types=[
        pltpu.VMEM((CH,), jnp.int32),
        pltpu.VMEM((CH,), jnp.int32),
        pltpu.VMEM((TEDGE,), jnp.int32),
        pltpu.VMEM((CH, D), jnp.float32),
        pltpu.VMEM_SHARED((N, D), jnp.float32),
        pltpu.SemaphoreType.DMA,
        pltpu.SemaphoreType.DMA,
        pltpu.SemaphoreType.DMA,
    ],
)
def _deg_kernel(dst_hbm, zeros_hbm, ones_hbm, out_hbm,
                didx0, didx1, didxt, ones_v, acc_sh, si0, si1, sit):
    c = lax.axis_index("c")
    s = lax.axis_index("s")
    wid = s * NC + c
    base = wid * EPW

    pltpu.async_copy(dst_hbm.at[pl.ds(base, CH)], didx0, si0)
    pltpu.async_copy(dst_hbm.at[pl.ds(base + CH, CH)], didx1, si1)
    pltpu.async_copy(dst_hbm.at[pl.ds(base + TOFF, TEDGE)], didxt, sit)
    pltpu.sync_copy(ones_hbm, ones_v)
    _zero_acc(zeros_hbm, acc_sh, s)
    plsc.subcore_barrier()

    def body(g, carry):
        a = 2 * g
        pltpu.make_async_copy(dst_hbm.at[pl.ds(base + a * CH, CH)],
                              didx0, si0).wait()
        pltpu.sync_copy(ones_v, acc_sh.at[didx0], add=True)

        @pl.when(g < NF // 2 - 1)
        def _():
            pltpu.async_copy(dst_hbm.at[pl.ds(base + (a + 2) * CH, CH)],
                             didx0, si0)

        pltpu.make_async_copy(dst_hbm.at[pl.ds(base + (a + 1) * CH, CH)],
                              didx1, si1).wait()
        pltpu.sync_copy(ones_v, acc_sh.at[didx1], add=True)

        @pl.when(g < NF // 2 - 1)
        def _():
            pltpu.async_copy(dst_hbm.at[pl.ds(base + (a + 3) * CH, CH)],
                             didx1, si1)

        return carry

    lax.fori_loop(0, NF // 2, body, 0)

    pltpu.make_async_copy(dst_hbm.at[pl.ds(base + TOFF, TEDGE)],
                          didxt, sit).wait()
    pltpu.sync_copy(ones_v.at[pl.ds(0, TEDGE)], acc_sh.at[didxt], add=True)

    plsc.subcore_barrier()
    _write_back(acc_sh, out_hbm, c, s)


assert NF % 2 == 0


@functools.partial(
    pl.kernel,
    out_type=jax.ShapeDtypeStruct((NC, N, D), jnp.float32),
    mesh=_mesh,
    scratch_types=[
        pltpu.VMEM((EPW,), jnp.int32),
        pltpu.VMEM((CH,), jnp.int32),
        pltpu.VMEM((CH,), jnp.int32),
        pltpu.VMEM((TEDGE,), jnp.int32),
        pltpu.VMEM((CH, D), jnp.float32),
        pltpu.VMEM((CH, D), jnp.float32),
        pltpu.VMEM((TEDGE, D), jnp.float32),
        pltpu.VMEM_SHARED((N, D), jnp.float32),
        pltpu.SemaphoreType.DMA,
        pltpu.SemaphoreType.DMA,
        pltpu.SemaphoreType.DMA,
        pltpu.SemaphoreType.DMA,
        pltpu.SemaphoreType.DMA,
        pltpu.SemaphoreType.DMA,
    ],
)
def _hop_kernel(g_hbm, src_hbm, dst_hbm, zeros_hbm, out_hbm,
                sidx_v, didx0, didx1, didxt, rows0, rows1, rowst, acc_sh,
                sg0, sg1, sgt, si0, si1, sit):
    c = lax.axis_index("c")
    s = lax.axis_index("s")
    wid = s * NC + c
    base = wid * EPW

    pltpu.sync_copy(src_hbm.at[pl.ds(base, EPW)], sidx_v)
    pltpu.async_copy(dst_hbm.at[pl.ds(base, CH)], didx0, si0)
    pltpu.async_copy(dst_hbm.at[pl.ds(base + CH, CH)], didx1, si1)
    pltpu.async_copy(dst_hbm.at[pl.ds(base + TOFF, TEDGE)], didxt, sit)
    pltpu.async_copy(g_hbm.at[sidx_v.at[pl.ds(0, CH)]], rows0, sg0)
    pltpu.async_copy(g_hbm.at[sidx_v.at[pl.ds(CH, CH)]], rows1, sg1)
    pltpu.async_copy(g_hbm.at[sidx_v.at[pl.ds(TOFF, TEDGE)]], rowst, sgt)

    _zero_acc(zeros_hbm, acc_sh, s)
    plsc.subcore_barrier()

    def body(g, carry):
        a = 2 * g
        pltpu.make_async_copy(dst_hbm.at[pl.ds(base + a * CH, CH)],
                              didx0, si0).wait()
        pltpu.make_async_copy(g_hbm.at[sidx_v.at[pl.ds(a * CH, CH)]],
                              rows0, sg0).wait()
        pltpu.sync_copy(rows0, acc_sh.at[didx0], add=True)

        @pl.when(g < NF // 2 - 1)
        def _():
            pltpu.async_copy(dst_hbm.at[pl.ds(base + (a + 2) * CH, CH)],
                             didx0, si0)
            pltpu.async_copy(g_hbm.at[sidx_v.at[pl.ds((a + 2) * CH, CH)]],
                             rows0, sg0)

        pltpu.make_async_copy(dst_hbm.at[pl.ds(base + (a + 1) * CH, CH)],
                              didx1, si1).wait()
        pltpu.make_async_copy(g_hbm.at[sidx_v.at[pl.ds((a + 1) * CH, CH)]],
                              rows1, sg1).wait()
        pltpu.sync_copy(rows1, acc_sh.at[didx1], add=True)

        @pl.when(g < NF // 2 - 1)
        def _():
            pltpu.async_copy(dst_hbm.at[pl.ds(base + (a + 3) * CH, CH)],
                             didx1, si1)
            pltpu.async_copy(g_hbm.at[sidx_v.at[pl.ds((a + 3) * CH, CH)]],
                             rows1, sg1)

        return carry

    lax.fori_loop(0, NF // 2, body, 0)

    pltpu.make_async_copy(dst_hbm.at[pl.ds(base + TOFF, TEDGE)],
                          didxt, sit).wait()
    pltpu.make_async_copy(g_hbm.at[sidx_v.at[pl.ds(TOFF, TEDGE)]],
                          rowst, sgt).wait()
    pltpu.sync_copy(rowst, acc_sh.at[didxt], add=True)

    plsc.subcore_barrier()
    _write_back(acc_sh, out_hbm, c, s)


_BR = 2000


def _norm_body(dacc_ref, feat_ref, g_ref, norm_ref):
    deg = dacc_ref[0, :, 0:1] + dacc_ref[1, :, 0:1]
    deg = jnp.maximum(deg, 1.0)
    nrm = lax.rsqrt(deg)
    norm_ref[...] = nrm
    g_ref[...] = feat_ref[...] * nrm


_norm_call = pl.pallas_call(
    _norm_body,
    grid=(N // _BR,),
    in_specs=[
        pl.BlockSpec((NC, _BR, D), lambda i: (0, i, 0)),
        pl.BlockSpec((_BR, D), lambda i: (i, 0)),
    ],
    out_specs=[
        pl.BlockSpec((_BR, D), lambda i: (i, 0)),
        pl.BlockSpec((_BR, 1), lambda i: (i, 0)),
    ],
    out_shape=[
        jax.ShapeDtypeStruct((N, D), jnp.float32),
        jax.ShapeDtypeStruct((N, 1), jnp.float32),
    ],
)


def _mid_body(p_ref, norm_ref, g_ref):
    nrm = norm_ref[...]
    g_ref[...] = (p_ref[0] + p_ref[1]) * (nrm * nrm)


_mid_call = pl.pallas_call(
    _mid_body,
    grid=(N // _BR,),
    in_specs=[
        pl.BlockSpec((NC, _BR, D), lambda i: (0, i, 0)),
        pl.BlockSpec((_BR, 1), lambda i: (i, 0)),
    ],
    out_specs=pl.BlockSpec((_BR, D), lambda i: (i, 0)),
    out_shape=jax.ShapeDtypeStruct((N, D), jnp.float32),
)


def _fin_body(q_ref, norm_ref, wt_ref, b_ref, out_ref):
    h = (q_ref[0] + q_ref[1]) * norm_ref[...]
    out_ref[...] = (
        jnp.dot(h, wt_ref[...], preferred_element_type=jnp.float32)
        + b_ref[...]
    )


_fin_call = pl.pallas_call(
    _fin_body,
    grid=(N // _BR,),
    in_specs=[
        pl.BlockSpec((NC, _BR, D), lambda i: (0, i, 0)),
        pl.BlockSpec((_BR, 1), lambda i: (i, 0)),
        pl.BlockSpec((D, D), lambda i: (0, 0)),
        pl.BlockSpec((1, D), lambda i: (0, 0)),
    ],
    out_specs=pl.BlockSpec((_BR, D), lambda i: (i, 0)),
    out_shape=jax.ShapeDtypeStruct((N, D), jnp.float32),
)


def kernel(feat, edge_index, W, b):
    ei = edge_index.astype(jnp.int32)
    src = ei[0]
    dst = ei[1]
    zeros = jnp.zeros((N, D), jnp.float32)
    ones = jnp.ones((CH, D), jnp.float32)

    dacc = _deg_kernel(dst, zeros, ones)
    g1, norm = _norm_call(dacc, feat)
    p = _hop_kernel(g1, src, dst, zeros)
    g2 = _mid_call(p, norm)
    q = _hop_kernel(g2, src, dst, zeros)
    out = _fin_call(q, norm, W.T.astype(jnp.float32), b.reshape(1, D))
    return out
